# baseline (speedup 1.0000x reference)
import functools

import jax
import jax.numpy as jnp
from jax import lax
from jax.experimental import pallas as pl
from jax.experimental.pallas import tpu as pltpu
from jax.experimental.pallas import tpu_sc as plsc

B = 4096
DIM_CIDX = 16
DIM_SLOT = 7
DPAD = 16


def _sc_geometry():
    try:
        info = plsc.get_sparse_core_info()
        return info.num_cores, info.num_subcores
    except Exception:
        return 2, 16


def _make_gather(nc, ns):
    nw = nc * ns
    bpw = B // nw
    mesh = plsc.VectorSubcoreMesh(
        core_axis_name="c", subcore_axis_name="s",
        num_cores=nc, num_subcores=ns)

    @functools.partial(
        pl.kernel,
        out_type=[
            jax.ShapeDtypeStruct((B, DIM_CIDX), jnp.float32),
            jax.ShapeDtypeStruct((B, DPAD), jnp.float32),
        ],
        mesh=mesh,
        compiler_params=pltpu.CompilerParams(use_tc_tiling_on_sc=False),
        scratch_types=[
            pltpu.VMEM((bpw,), jnp.int32),
            pltpu.VMEM((bpw,), jnp.int32),
            pltpu.VMEM((bpw, DIM_CIDX), jnp.float32),
            pltpu.VMEM((bpw, DPAD), jnp.float32),
            pltpu.SemaphoreType.DMA,
            pltpu.SemaphoreType.DMA,
        ],
    )
    def gather_kernel(cidx_hbm, slot_hbm, ctab_hbm, stab_hbm,
                      c_out, s_out, ci_v, si_v, crows, srows, sem_c, sem_s):
        wid = lax.axis_index("s") * nc + lax.axis_index("c")
        base = wid * bpw
        pltpu.sync_copy(cidx_hbm.at[pl.ds(base, bpw)], ci_v)
        pltpu.sync_copy(slot_hbm.at[pl.ds(base, bpw)], si_v)
        cp_c = pltpu.async_copy(ctab_hbm.at[ci_v], crows, sem_c)
        cp_s = pltpu.async_copy(stab_hbm.at[si_v], srows, sem_s)
        cp_c.wait()
        cp_s.wait()
        pltpu.sync_copy(crows, c_out.at[pl.ds(base, bpw)])
        pltpu.sync_copy(srows, s_out.at[pl.ds(base, bpw)])

    return gather_kernel


def _mlp_body(c_ref, s_ref, f_ref, a1_ref, a2_ref, a3_ref, b1_ref,
              w2t_ref, b2_ref, w3t_ref, b3_ref, o_ref):
    h = (jnp.dot(c_ref[...], a1_ref[...], preferred_element_type=jnp.float32)
         + jnp.dot(s_ref[...], a2_ref[...], preferred_element_type=jnp.float32)
         + f_ref[...] * a3_ref[...]
         + b1_ref[...])
    h = jnp.maximum(h, 0.0)
    h = jnp.dot(h, w2t_ref[...], preferred_element_type=jnp.float32) + b2_ref[...]
    h = jnp.maximum(h, 0.0)
    o_ref[...] = (jnp.dot(h, w3t_ref[...], preferred_element_type=jnp.float32)
                  + b3_ref[...])


def kernel(x, emb_cidx, emb_slot, W1, b1, W2, b2, W3, b3):
    cidx = x[:, 0]
    slot = x[:, 1]
    xf = x[:, 2].astype(jnp.float32).reshape(B, 1)

    stab = jnp.concatenate(
        [emb_slot, jnp.zeros((emb_slot.shape[0], DPAD - DIM_SLOT), jnp.float32)],
        axis=1)

    nc, ns = _sc_geometry()
    c_rows, s_rows = _make_gather(nc, ns)(cidx, slot, emb_cidx, stab)

    w1t = W1.T
    a1 = w1t[:DIM_CIDX]
    a2 = jnp.zeros((DPAD, 24), jnp.float32).at[:DIM_SLOT].set(
        w1t[DIM_CIDX:DIM_CIDX + DIM_SLOT])
    a3 = w1t[DIM_CIDX + DIM_SLOT:]

    y = pl.pallas_call(
        _mlp_body,
        out_shape=jax.ShapeDtypeStruct((B, 1), jnp.float32),
    )(c_rows, s_rows, xf, a1, a2, a3, b1.reshape(1, 24),
      W2.T, b2.reshape(1, 12), W3.T, b3.reshape(1, 1))
    return y.reshape(-1)
